# Initial kernel scaffold; baseline (speedup 1.0000x reference)
import jax
import jax.numpy as jnp
from jax.experimental import pallas as pl


def kernel(x, edge_index, edge_attr, batch, W_proj, b_proj, W_edge, b_edge, W_root, b_conv, W_ih, W_hh, b_ih, b_hh, W_cls, b_cls, prelu_a, W_sp, b_sp, W_y1, b_y1, W_y2, b_y2):
    raise NotImplementedError("write your pallas kernel here")


# speedup vs baseline: 1.5182x; 1.5182x over previous
import functools

import jax
import jax.numpy as jnp
from jax import lax
from jax.experimental import pallas as pl
from jax.experimental.pallas import tpu as pltpu
from jax.experimental.pallas import tpu_sc as plsc

N = 10000
E = 160000
H = 32
G = 64
TN = 1000
TE = 2000
CH = 128
NCHUNK = E // CH
NW = 32
JMAX = (NCHUNK + NW - 1) // NW

_f32 = jnp.float32


def _sigm(t):
    return 1.0 / (1.0 + jnp.exp(-t))


def _tanh(t):
    return 2.0 / (1.0 + jnp.exp(-2.0 * t)) - 1.0


def _h0_body(x_ref, wp_ref, bp_ref, o_ref):
    o_ref[...] = jnp.maximum(
        jnp.dot(x_ref[...], wp_ref[...], preferred_element_type=_f32)
        + bp_ref[...], 0.0)


def _h0(x, W_proj, b_proj):
    d_in = x.shape[1]
    return pl.pallas_call(
        _h0_body,
        grid=(N // TN,),
        in_specs=[
            pl.BlockSpec((TN, d_in), lambda i: (i, 0)),
            pl.BlockSpec((d_in, H), lambda i: (0, 0)),
            pl.BlockSpec((1, H), lambda i: (0, 0)),
        ],
        out_specs=pl.BlockSpec((TN, H), lambda i: (i, 0)),
        out_shape=jax.ShapeDtypeStruct((N, H), _f32),
    )(x, W_proj, b_proj.reshape(1, H))


def _msg_body(s_ref, a_ref, wh_ref, b_ref, o_ref):
    s = s_ref[...]
    a_bf = a_ref[...].astype(jnp.bfloat16)
    s_bf = s.astype(jnp.bfloat16).astype(_f32)
    acc = jnp.zeros_like(s)
    for h in range(H):
        ew_h = (jnp.dot(a_bf, wh_ref[h], preferred_element_type=_f32)
                + b_ref[h:h + 1, :])
        ew_h = ew_h.astype(jnp.bfloat16).astype(_f32)
        acc += s_bf[:, h:h + 1] * ew_h
    o_ref[...] = acc


def _messages(s, edge_attr, Wh_bf, B):
    de = edge_attr.shape[1]
    return pl.pallas_call(
        _msg_body,
        grid=(E // TE,),
        in_specs=[
            pl.BlockSpec((TE, H), lambda i: (i, 0)),
            pl.BlockSpec((TE, de), lambda i: (i, 0)),
            pl.BlockSpec((H, de, H), lambda i: (0, 0, 0)),
            pl.BlockSpec((H, H), lambda i: (0, 0)),
        ],
        out_specs=pl.BlockSpec((TE, H), lambda i: (i, 0)),
        out_shape=jax.ShapeDtypeStruct((E, H), _f32),
    )(s, edge_attr, Wh_bf, B)


def _upd_body(p0_ref, p1_ref, h_ref, wr_ref, bc_ref,
              wir_ref, wiz_ref, win_ref, whr_ref, whz_ref, whn_ref,
              bir_ref, biz_ref, bin_ref, bhr_ref, bhz_ref, bhn_ref,
              o_ref):
    h = h_ref[...]
    aggr = p0_ref[0] + p1_ref[0]
    m = jnp.maximum(
        aggr + jnp.dot(h, wr_ref[...], preferred_element_type=_f32)
        + bc_ref[...], 0.0)

    def mm(v, w_ref, b_ref):
        return jnp.dot(v, w_ref[...], preferred_element_type=_f32) + b_ref[...]

    r = _sigm(mm(m, wir_ref, bir_ref) + mm(h, whr_ref, bhr_ref))
    z = _sigm(mm(m, wiz_ref, biz_ref) + mm(h, whz_ref, bhz_ref))
    n = _tanh(mm(m, win_ref, bin_ref) + r * mm(h, whn_ref, bhn_ref))
    o_ref[...] = (1.0 - z) * n + z * h


def _update(parts, h, W_root, b_conv, gru_w):
    row = pl.BlockSpec((TN, H), lambda i: (i, 0))
    whh = pl.BlockSpec((H, H), lambda i: (0, 0))
    b1h = pl.BlockSpec((1, H), lambda i: (0, 0))
    return pl.pallas_call(
        _upd_body,
        grid=(N // TN,),
        in_specs=[
            pl.BlockSpec((1, TN, H), lambda i: (0, i, 0)),
            pl.BlockSpec((1, TN, H), lambda i: (1, i, 0)),
            row, whh, b1h,
            whh, whh, whh, whh, whh, whh,
            b1h, b1h, b1h, b1h, b1h, b1h,
        ],
        out_specs=row,
        out_shape=jax.ShapeDtypeStruct((N, H), _f32),
    )(parts, parts, h, W_root, b_conv.reshape(1, H), *gru_w)


def _read_body(h0_ref, nf_ref, b_ref, wcls_ref, bcls_ref, al_ref,
               wsp_ref, bsp_ref, wy1_ref, by1_ref, wy2_ref, by2_ref,
               pb_ref, y_ref, s0_ref, s1_ref, cnt_ref):
    i = pl.program_id(0)

    @pl.when(i == 0)
    def _():
        s0_ref[...] = jnp.zeros_like(s0_ref)
        s1_ref[...] = jnp.zeros_like(s1_ref)
        cnt_ref[...] = jnp.zeros_like(cnt_ref)

    h0 = h0_ref[...]
    nf = nf_ref[...]
    b = b_ref[...]
    gid = lax.broadcasted_iota(jnp.int32, (TN, G), 1)
    onehot = (b == gid).astype(_f32)
    dn = (((0,), (0,)), ((), ()))
    s0_ref[...] += lax.dot_general(onehot, h0, dn, preferred_element_type=_f32, precision=lax.Precision.HIGHEST)
    s1_ref[...] += lax.dot_general(onehot, nf, dn, preferred_element_type=_f32, precision=lax.Precision.HIGHEST)
    cnt_ref[...] += jnp.sum(onehot, axis=0, keepdims=True)

    pb_ref[...] = (jnp.dot(nf, wcls_ref[...], preferred_element_type=_f32)
                   + bcls_ref[...])

    @pl.when(i == (N // TN) - 1)
    def _():
        cnt = jnp.maximum(cnt_ref[...], 1.0)
        inv = (1.0 / cnt).reshape(G, 1)
        r0 = s0_ref[...] * inv
        r1 = s1_ref[...] * inv
        sp = (lax.dot_general(r0, wsp_ref[...][:H, :], (((1,), (0,)), ((), ())),
                              preferred_element_type=_f32)
              + lax.dot_general(r1, wsp_ref[...][H:, :], (((1,), (0,)), ((), ())),
                                preferred_element_type=_f32)
              + bsp_ref[...])
        a = al_ref[0, 0]
        sp = jnp.where(sp >= 0.0, sp, a * sp)
        yh = jnp.maximum(
            jnp.dot(sp, wy1_ref[...], preferred_element_type=_f32)
            + by1_ref[...], 0.0)
        y_ref[...] = (jnp.dot(yh, wy2_ref[...], preferred_element_type=_f32)
                      + by2_ref[...])


def _readout(h0, nf, batch, W_cls, b_cls, prelu_a, W_sp, b_sp,
             W_y1, b_y1, W_y2, b_y2):
    rf = W_sp.shape[1]
    hid = W_y1.shape[1]
    row = pl.BlockSpec((TN, H), lambda i: (i, 0))
    full = lambda *shape: pl.BlockSpec(shape, lambda i: tuple(0 for _ in shape))
    pb, y = pl.pallas_call(
        _read_body,
        grid=(N // TN,),
        in_specs=[
            row, row,
            pl.BlockSpec((TN, 1), lambda i: (i, 0)),
            full(H, 1), full(1, 1), full(1, 1),
            full(2 * H, rf), full(1, rf),
            full(rf, hid), full(1, hid),
            full(hid, 1), full(1, 1),
        ],
        out_specs=[
            pl.BlockSpec((TN, 1), lambda i: (i, 0)),
            pl.BlockSpec((G, 1), lambda i: (0, 0)),
        ],
        out_shape=[
            jax.ShapeDtypeStruct((N, 1), _f32),
            jax.ShapeDtypeStruct((G, 1), _f32),
        ],
        scratch_shapes=[
            pltpu.VMEM((G, H), _f32),
            pltpu.VMEM((G, H), _f32),
            pltpu.VMEM((1, G), _f32),
        ],
    )(h0, nf, batch.reshape(N, 1), W_cls, b_cls.reshape(1, 1),
      prelu_a.reshape(1, 1), W_sp, b_sp.reshape(1, rf),
      W_y1, b_y1.reshape(1, hid), W_y2, b_y2.reshape(1, 1))
    return pb[:, 0], y[:, 0]


def _sc_mesh():
    return plsc.VectorSubcoreMesh(core_axis_name="c", subcore_axis_name="s")


@functools.partial(
    pl.kernel,
    mesh=_sc_mesh(),
    out_type=jax.ShapeDtypeStruct((E, H), _f32),
    scratch_types=[
        pltpu.VMEM((CH,), jnp.int32),
        pltpu.VMEM((CH, H), _f32),
        pltpu.SemaphoreType.DMA,
    ],
    compiler_params=pltpu.CompilerParams(use_tc_tiling_on_sc=False),
)
def _sc_gather(nf_hbm, src_hbm, out_hbm, idx_v, rows_v, sem):
    wid = lax.axis_index("s") * 2 + lax.axis_index("c")

    def body(j, carry):
        c = wid + NW * j

        @pl.when(c < NCHUNK)
        def _():
            pltpu.sync_copy(src_hbm.at[c], idx_v)
            pltpu.async_copy(nf_hbm.at[idx_v], rows_v, sem).wait()
            pltpu.sync_copy(rows_v, out_hbm.at[pl.ds(c * CH, CH)])

        return carry

    lax.fori_loop(0, JMAX, body, 0)


@functools.partial(
    pl.kernel,
    mesh=_sc_mesh(),
    out_type=jax.ShapeDtypeStruct((2, N, H), _f32),
    scratch_types=[
        pltpu.VMEM((CH,), jnp.int32),
        pltpu.VMEM((CH, H), _f32),
        pltpu.VMEM_SHARED((N, H), _f32),
        pltpu.SemaphoreType.DMA,
    ],
    compiler_params=pltpu.CompilerParams(use_tc_tiling_on_sc=False),
)
def _sc_scatter(msg_hbm, dst_hbm, zero_hbm, out_hbm, idx_v, msg_v, acc, sem):
    cid = lax.axis_index("c")
    sid = lax.axis_index("s")
    wid = sid * 2 + cid
    rows_per_tile = N // 16

    pltpu.sync_copy(zero_hbm.at[pl.ds(sid * rows_per_tile, rows_per_tile)],
                    acc.at[pl.ds(sid * rows_per_tile, rows_per_tile)])
    plsc.subcore_barrier()

    def body(j, carry):
        c = wid + NW * j

        @pl.when(c < NCHUNK)
        def _():
            pltpu.sync_copy(dst_hbm.at[c], idx_v)
            pltpu.sync_copy(msg_hbm.at[pl.ds(c * CH, CH)], msg_v)
            pltpu.sync_copy(msg_v, acc.at[idx_v], add=True)

        return carry

    lax.fori_loop(0, JMAX, body, 0)
    plsc.subcore_barrier()
    pltpu.sync_copy(acc.at[pl.ds(sid * rows_per_tile, rows_per_tile)],
                    out_hbm.at[cid, pl.ds(sid * rows_per_tile, rows_per_tile)])


def kernel(x, edge_index, edge_attr, batch, W_proj, b_proj, W_edge, b_edge,
           W_root, b_conv, W_ih, W_hh, b_ih, b_hh, W_cls, b_cls, prelu_a,
           W_sp, b_sp, W_y1, b_y1, W_y2, b_y2):
    de = edge_attr.shape[1]
    src = edge_index[0].reshape(NCHUNK, CH)
    dst = edge_index[1].reshape(NCHUNK, CH)
    Wh_bf = W_edge.reshape(de, H, H).transpose(1, 0, 2).astype(jnp.bfloat16)
    B = b_edge.reshape(H, H)
    zero = jnp.zeros((N, H), _f32)
    gru_w = (
        W_ih[0:H].T, W_ih[H:2 * H].T, W_ih[2 * H:].T,
        W_hh[0:H].T, W_hh[H:2 * H].T, W_hh[2 * H:].T,
        b_ih[0:H].reshape(1, H), b_ih[H:2 * H].reshape(1, H),
        b_ih[2 * H:].reshape(1, H),
        b_hh[0:H].reshape(1, H), b_hh[H:2 * H].reshape(1, H),
        b_hh[2 * H:].reshape(1, H),
    )

    h0 = _h0(x, W_proj, b_proj)
    h = h0
    for _ in range(2):
        s = _sc_gather(h, src)
        msg = _messages(s, edge_attr, Wh_bf, B)
        parts = _sc_scatter(msg, dst, zero)
        h = _update(parts, h, W_root, b_conv, gru_w)

    return _readout(h0, h, batch, W_cls, b_cls, prelu_a,
                    W_sp, b_sp, W_y1, b_y1, W_y2, b_y2)


# speedup vs baseline: 2.7381x; 1.8036x over previous
import functools

import jax
import jax.numpy as jnp
from jax import lax
from jax.experimental import pallas as pl
from jax.experimental.pallas import tpu as pltpu
from jax.experimental.pallas import tpu_sc as plsc

N = 10000
E = 160000
H = 32
G = 64
TN = 1000
TE = 2000
TEM = 800
CH = 128
NCHUNK = E // CH
NW = 32
JMAX = (NCHUNK + NW - 1) // NW

_f32 = jnp.float32


def _sigm(t):
    return 1.0 / (1.0 + jnp.exp(-t))


def _tanh(t):
    return 2.0 / (1.0 + jnp.exp(-2.0 * t)) - 1.0


def _h0_body(x_ref, wp_ref, bp_ref, o_ref):
    o_ref[...] = jnp.maximum(
        jnp.dot(x_ref[...], wp_ref[...], preferred_element_type=_f32)
        + bp_ref[...], 0.0)


def _h0(x, W_proj, b_proj):
    d_in = x.shape[1]
    return pl.pallas_call(
        _h0_body,
        grid=(N // TN,),
        in_specs=[
            pl.BlockSpec((TN, d_in), lambda i: (i, 0)),
            pl.BlockSpec((d_in, H), lambda i: (0, 0)),
            pl.BlockSpec((1, H), lambda i: (0, 0)),
        ],
        out_specs=pl.BlockSpec((TN, H), lambda i: (i, 0)),
        out_shape=jax.ShapeDtypeStruct((N, H), _f32),
    )(x, W_proj, b_proj.reshape(1, H))


def _msg_body(s_ref, a_ref, w17_ref, sel_ref, o_ref):
    a_bf = a_ref[...].astype(jnp.bfloat16).astype(_f32)
    ones = jnp.ones((TEM, 1), _f32)
    a17 = jnp.concatenate([a_bf, ones], axis=1)
    ewp = jnp.dot(a17, w17_ref[...], preferred_element_type=_f32)
    ewp = ewp.astype(jnp.bfloat16).astype(_f32)
    s_bf = s_ref[...].astype(jnp.bfloat16).astype(_f32)
    s_t = jnp.concatenate([s_bf] * H, axis=1)
    o_ref[...] = jnp.dot(ewp * s_t, sel_ref[...],
                         preferred_element_type=_f32)


def _messages(s, edge_attr, W17, Sel):
    de = edge_attr.shape[1]
    return pl.pallas_call(
        _msg_body,
        grid=(E // TEM,),
        in_specs=[
            pl.BlockSpec((TEM, H), lambda i: (i, 0)),
            pl.BlockSpec((TEM, de), lambda i: (i, 0)),
            pl.BlockSpec((de + 1, H * H), lambda i: (0, 0)),
            pl.BlockSpec((H * H, H), lambda i: (0, 0)),
        ],
        out_specs=pl.BlockSpec((TEM, H), lambda i: (i, 0)),
        out_shape=jax.ShapeDtypeStruct((E, H), _f32),
    )(s, edge_attr, W17, Sel)


def _upd_body(p0_ref, p1_ref, h_ref, wr_ref, bc_ref,
              wir_ref, wiz_ref, win_ref, whr_ref, whz_ref, whn_ref,
              bir_ref, biz_ref, bin_ref, bhr_ref, bhz_ref, bhn_ref,
              o_ref):
    h = h_ref[...]
    aggr = p0_ref[0] + p1_ref[0]
    m = jnp.maximum(
        aggr + jnp.dot(h, wr_ref[...], preferred_element_type=_f32)
        + bc_ref[...], 0.0)

    def mm(v, w_ref, b_ref):
        return jnp.dot(v, w_ref[...], preferred_element_type=_f32) + b_ref[...]

    r = _sigm(mm(m, wir_ref, bir_ref) + mm(h, whr_ref, bhr_ref))
    z = _sigm(mm(m, wiz_ref, biz_ref) + mm(h, whz_ref, bhz_ref))
    n = _tanh(mm(m, win_ref, bin_ref) + r * mm(h, whn_ref, bhn_ref))
    o_ref[...] = (1.0 - z) * n + z * h


def _update(parts, h, W_root, b_conv, gru_w):
    row = pl.BlockSpec((TN, H), lambda i: (i, 0))
    whh = pl.BlockSpec((H, H), lambda i: (0, 0))
    b1h = pl.BlockSpec((1, H), lambda i: (0, 0))
    return pl.pallas_call(
        _upd_body,
        grid=(N // TN,),
        in_specs=[
            pl.BlockSpec((1, TN, H), lambda i: (0, i, 0)),
            pl.BlockSpec((1, TN, H), lambda i: (1, i, 0)),
            row, whh, b1h,
            whh, whh, whh, whh, whh, whh,
            b1h, b1h, b1h, b1h, b1h, b1h,
        ],
        out_specs=row,
        out_shape=jax.ShapeDtypeStruct((N, H), _f32),
    )(parts, parts, h, W_root, b_conv.reshape(1, H), *gru_w)


def _read_body(h0_ref, nf_ref, b_ref, wcls_ref, bcls_ref, al_ref,
               wsp_ref, bsp_ref, wy1_ref, by1_ref, wy2_ref, by2_ref,
               pb_ref, y_ref, s0_ref, s1_ref, cnt_ref):
    i = pl.program_id(0)

    @pl.when(i == 0)
    def _():
        s0_ref[...] = jnp.zeros_like(s0_ref)
        s1_ref[...] = jnp.zeros_like(s1_ref)
        cnt_ref[...] = jnp.zeros_like(cnt_ref)

    h0 = h0_ref[...]
    nf = nf_ref[...]
    b = b_ref[...]
    gid = lax.broadcasted_iota(jnp.int32, (TN, G), 1)
    onehot = (b == gid).astype(_f32)
    dn = (((0,), (0,)), ((), ()))
    s0_ref[...] += lax.dot_general(onehot, h0, dn, preferred_element_type=_f32, precision=lax.Precision.HIGHEST)
    s1_ref[...] += lax.dot_general(onehot, nf, dn, preferred_element_type=_f32, precision=lax.Precision.HIGHEST)
    cnt_ref[...] += jnp.sum(onehot, axis=0, keepdims=True)

    pb_ref[...] = (jnp.dot(nf, wcls_ref[...], preferred_element_type=_f32)
                   + bcls_ref[...])

    @pl.when(i == (N // TN) - 1)
    def _():
        cnt = jnp.maximum(cnt_ref[...], 1.0)
        inv = (1.0 / cnt).reshape(G, 1)
        r0 = s0_ref[...] * inv
        r1 = s1_ref[...] * inv
        sp = (lax.dot_general(r0, wsp_ref[...][:H, :], (((1,), (0,)), ((), ())),
                              preferred_element_type=_f32)
              + lax.dot_general(r1, wsp_ref[...][H:, :], (((1,), (0,)), ((), ())),
                                preferred_element_type=_f32)
              + bsp_ref[...])
        a = al_ref[0, 0]
        sp = jnp.where(sp >= 0.0, sp, a * sp)
        yh = jnp.maximum(
            jnp.dot(sp, wy1_ref[...], preferred_element_type=_f32)
            + by1_ref[...], 0.0)
        y_ref[...] = (jnp.dot(yh, wy2_ref[...], preferred_element_type=_f32)
                      + by2_ref[...])


def _readout(h0, nf, batch, W_cls, b_cls, prelu_a, W_sp, b_sp,
             W_y1, b_y1, W_y2, b_y2):
    rf = W_sp.shape[1]
    hid = W_y1.shape[1]
    row = pl.BlockSpec((TN, H), lambda i: (i, 0))
    full = lambda *shape: pl.BlockSpec(shape, lambda i: tuple(0 for _ in shape))
    pb, y = pl.pallas_call(
        _read_body,
        grid=(N // TN,),
        in_specs=[
            row, row,
            pl.BlockSpec((TN, 1), lambda i: (i, 0)),
            full(H, 1), full(1, 1), full(1, 1),
            full(2 * H, rf), full(1, rf),
            full(rf, hid), full(1, hid),
            full(hid, 1), full(1, 1),
        ],
        out_specs=[
            pl.BlockSpec((TN, 1), lambda i: (i, 0)),
            pl.BlockSpec((G, 1), lambda i: (0, 0)),
        ],
        out_shape=[
            jax.ShapeDtypeStruct((N, 1), _f32),
            jax.ShapeDtypeStruct((G, 1), _f32),
        ],
        scratch_shapes=[
            pltpu.VMEM((G, H), _f32),
            pltpu.VMEM((G, H), _f32),
            pltpu.VMEM((1, G), _f32),
        ],
    )(h0, nf, batch.reshape(N, 1), W_cls, b_cls.reshape(1, 1),
      prelu_a.reshape(1, 1), W_sp, b_sp.reshape(1, rf),
      W_y1, b_y1.reshape(1, hid), W_y2, b_y2.reshape(1, 1))
    return pb[:, 0], y[:, 0]


def _sc_mesh():
    return plsc.VectorSubcoreMesh(core_axis_name="c", subcore_axis_name="s")


@functools.partial(
    pl.kernel,
    mesh=_sc_mesh(),
    out_type=jax.ShapeDtypeStruct((E, H), _f32),
    scratch_types=[
        pltpu.VMEM((CH,), jnp.int32),
        pltpu.VMEM((CH, H), _f32),
        pltpu.SemaphoreType.DMA,
    ],
    compiler_params=pltpu.CompilerParams(use_tc_tiling_on_sc=False),
)
def _sc_gather(nf_hbm, src_hbm, out_hbm, idx_v, rows_v, sem):
    wid = lax.axis_index("s") * 2 + lax.axis_index("c")

    def body(j, carry):
        c = wid + NW * j

        @pl.when(c < NCHUNK)
        def _():
            pltpu.sync_copy(src_hbm.at[c], idx_v)
            pltpu.async_copy(nf_hbm.at[idx_v], rows_v, sem).wait()
            pltpu.sync_copy(rows_v, out_hbm.at[pl.ds(c * CH, CH)])

        return carry

    lax.fori_loop(0, JMAX, body, 0)


@functools.partial(
    pl.kernel,
    mesh=_sc_mesh(),
    out_type=jax.ShapeDtypeStruct((2, N, H), _f32),
    scratch_types=[
        pltpu.VMEM((CH,), jnp.int32),
        pltpu.VMEM((CH, H), _f32),
        pltpu.VMEM_SHARED((N, H), _f32),
        pltpu.SemaphoreType.DMA,
    ],
    compiler_params=pltpu.CompilerParams(use_tc_tiling_on_sc=False),
)
def _sc_scatter(msg_hbm, dst_hbm, zero_hbm, out_hbm, idx_v, msg_v, acc, sem):
    cid = lax.axis_index("c")
    sid = lax.axis_index("s")
    wid = sid * 2 + cid
    rows_per_tile = N // 16

    pltpu.sync_copy(zero_hbm.at[pl.ds(sid * rows_per_tile, rows_per_tile)],
                    acc.at[pl.ds(sid * rows_per_tile, rows_per_tile)])
    plsc.subcore_barrier()

    def body(j, carry):
        c = wid + NW * j

        @pl.when(c < NCHUNK)
        def _():
            pltpu.sync_copy(dst_hbm.at[c], idx_v)
            pltpu.sync_copy(msg_hbm.at[pl.ds(c * CH, CH)], msg_v)
            pltpu.sync_copy(msg_v, acc.at[idx_v], add=True)

        return carry

    lax.fori_loop(0, JMAX, body, 0)
    plsc.subcore_barrier()
    pltpu.sync_copy(acc.at[pl.ds(sid * rows_per_tile, rows_per_tile)],
                    out_hbm.at[cid, pl.ds(sid * rows_per_tile, rows_per_tile)])


def kernel(x, edge_index, edge_attr, batch, W_proj, b_proj, W_edge, b_edge,
           W_root, b_conv, W_ih, W_hh, b_ih, b_hh, W_cls, b_cls, prelu_a,
           W_sp, b_sp, W_y1, b_y1, W_y2, b_y2):
    de = edge_attr.shape[1]
    src = edge_index[0].reshape(NCHUNK, CH)
    dst = edge_index[1].reshape(NCHUNK, CH)
    Wp = W_edge.reshape(de, H, H).transpose(0, 2, 1).reshape(de, H * H)
    Wp = Wp.astype(jnp.bfloat16).astype(_f32)
    bp = b_edge.reshape(H, H).T.reshape(1, H * H)
    W17 = jnp.concatenate([Wp, bp], axis=0)
    Sel = jnp.repeat(jnp.eye(H, dtype=_f32), H, axis=0)
    zero = jnp.zeros((N, H), _f32)
    gru_w = (
        W_ih[0:H].T, W_ih[H:2 * H].T, W_ih[2 * H:].T,
        W_hh[0:H].T, W_hh[H:2 * H].T, W_hh[2 * H:].T,
        b_ih[0:H].reshape(1, H), b_ih[H:2 * H].reshape(1, H),
        b_ih[2 * H:].reshape(1, H),
        b_hh[0:H].reshape(1, H), b_hh[H:2 * H].reshape(1, H),
        b_hh[2 * H:].reshape(1, H),
    )

    h0 = _h0(x, W_proj, b_proj)
    h = h0
    for _ in range(2):
        s = _sc_gather(h, src)
        msg = _messages(s, edge_attr, W17, Sel)
        parts = _sc_scatter(msg, dst, zero)
        h = _update(parts, h, W_root, b_conv, gru_w)

    return _readout(h0, h, batch, W_cls, b_cls, prelu_a,
                    W_sp, b_sp, W_y1, b_y1, W_y2, b_y2)
